# Initial kernel scaffold; baseline (speedup 1.0000x reference)
import jax
import jax.numpy as jnp
from jax.experimental import pallas as pl


def kernel(x, edge_index, attention, W1, b1, W2, b2):
    raise NotImplementedError("write your pallas kernel here")


# speedup vs baseline: 4.1192x; 4.1192x over previous
import functools

import jax
import jax.numpy as jnp
from jax import lax
from jax.experimental import pallas as pl
from jax.experimental.pallas import tpu as pltpu
from jax.experimental.pallas import tpu_sc as plsc

N = 10000
D = 128
E = 320000

NC = 2
NS = 16
NW = NC * NS

CHUNK = 128
CPW = -(-E // (NW * CHUNK))
EPW = CPW * CHUNK
E_PAD = EPW * NW

ROWS_PER_SUB = N // NS
ZROWS = 125


@functools.partial(
    pl.kernel,
    out_type=jax.ShapeDtypeStruct((NC, NS, ROWS_PER_SUB, D), jnp.float32),
    mesh=plsc.VectorSubcoreMesh(core_axis_name="c", subcore_axis_name="s"),
    scratch_types=[
        pltpu.VMEM((CHUNK,), jnp.int32),
        pltpu.VMEM((CHUNK,), jnp.int32),
        pltpu.VMEM((CHUNK,), jnp.float32),
        pltpu.VMEM((CHUNK, D), jnp.float32),
        pltpu.VMEM((ZROWS, D), jnp.float32),
        pltpu.VMEM_SHARED((N, D), jnp.float32),
        pltpu.SemaphoreType.DMA,
    ],
)
def _sc_aggregate(x_hbm, src_hbm, dst_hbm, att_hbm, out_hbm,
                  src_v, dst_v, att_v, rows_v, zero_v, ego_sh, sem):
    c = lax.axis_index("c")
    s = lax.axis_index("s")
    wid = s * NC + c

    zvec = jnp.zeros((16,), jnp.float32)

    def _zero_row(i, _):
        r = i // 8
        j = i % 8
        zero_v[r, pl.ds(j * 16, 16)] = zvec
        return 0

    lax.fori_loop(0, ZROWS * 8, _zero_row, 0)
    stripe = s * ROWS_PER_SUB
    for k in range(ROWS_PER_SUB // ZROWS):
        pltpu.sync_copy(zero_v, ego_sh.at[pl.ds(stripe + k * ZROWS, ZROWS)])
    plsc.subcore_barrier()

    def _chunk(i, _):
        base = wid * EPW + i * CHUNK
        pltpu.sync_copy(src_hbm.at[pl.ds(base, CHUNK)], src_v)
        pltpu.sync_copy(dst_hbm.at[pl.ds(base, CHUNK)], dst_v)
        pltpu.sync_copy(att_hbm.at[pl.ds(base, CHUNK)], att_v)
        pltpu.async_copy(x_hbm.at[src_v], rows_v, sem).wait()

        def _group(g, _):
            a16 = att_v[pl.ds(g * 16, 16)]

            def _edge(t, _):
                spl = jnp.take_along_axis(a16, jnp.full((16,), t, jnp.int32),
                                          axis=0)
                e = g * 16 + t
                for j in range(D // 16):
                    rows_v[e, pl.ds(j * 16, 16)] = (
                        rows_v[e, pl.ds(j * 16, 16)] * spl)
                return 0

            lax.fori_loop(0, 16, _edge, 0)
            return 0

        lax.fori_loop(0, CHUNK // 16, _group, 0)
        pltpu.sync_copy(rows_v, ego_sh.at[dst_v], add=True)
        return 0

    lax.fori_loop(0, CPW, _chunk, 0)
    plsc.subcore_barrier()

    pltpu.sync_copy(ego_sh.at[pl.ds(stripe, ROWS_PER_SUB)], out_hbm.at[c, s])


BLK = 1000


def _mlp_body(x_ref, p0_ref, p1_ref, w1_ref, b1_ref, w2_ref, b2_ref, o_ref):
    ego = p0_ref[...] + p1_ref[...]
    xv = x_ref[...]
    h1 = jnp.dot(xv + ego, w1_ref[...], preferred_element_type=jnp.float32) + b1_ref[...]
    h2 = jnp.dot(xv * ego, w2_ref[...], preferred_element_type=jnp.float32) + b2_ref[...]
    o_ref[...] = (jnp.where(h1 >= 0, h1, 0.01 * h1)
                  + jnp.where(h2 >= 0, h2, 0.01 * h2))


def _mlp(x, partials, W1, b1, W2, b2):
    grid = N // BLK
    return pl.pallas_call(
        _mlp_body,
        grid=(grid,),
        in_specs=[
            pl.BlockSpec((BLK, D), lambda i: (i, 0)),
            pl.BlockSpec((BLK, D), lambda i: (i, 0)),
            pl.BlockSpec((BLK, D), lambda i: (i + N // BLK, 0)),
            pl.BlockSpec((D, D), lambda i: (0, 0)),
            pl.BlockSpec((1, D), lambda i: (0, 0)),
            pl.BlockSpec((D, D), lambda i: (0, 0)),
            pl.BlockSpec((1, D), lambda i: (0, 0)),
        ],
        out_specs=pl.BlockSpec((BLK, D), lambda i: (i, 0)),
        out_shape=jax.ShapeDtypeStruct((N, D), jnp.float32),
    )(x, partials, partials, W1, b1, W2, b2)


def kernel(x, edge_index, attention, W1, b1, W2, b2):
    src = edge_index[0].astype(jnp.int32)
    dst = edge_index[1].astype(jnp.int32)
    pad = E_PAD - E
    src = jnp.concatenate([src, jnp.zeros((pad,), jnp.int32)])
    dst = jnp.concatenate([dst, jnp.zeros((pad,), jnp.int32)])
    att = jnp.concatenate([attention, jnp.zeros((pad,), jnp.float32)])
    partials = _sc_aggregate(x, src, dst, att).reshape(NC * N, D)
    return _mlp(x, partials, W1, b1.reshape(1, D), W2, b2.reshape(1, D))


# speedup vs baseline: 4.4111x; 1.0709x over previous
import functools

import jax
import jax.numpy as jnp
from jax import lax
from jax.experimental import pallas as pl
from jax.experimental.pallas import tpu as pltpu
from jax.experimental.pallas import tpu_sc as plsc

N = 10000
D = 128
E = 320000

NC = 2
NS = 16
NW = NC * NS

CHUNK = 128
CPW = 80
EPW = CPW * CHUNK
E_PAD = EPW * NW

ROWS_PER_SUB = N // NS
ZROWS = 125
PH = 2
CPP = CPW // PH


@functools.partial(
    pl.kernel,
    out_type=jax.ShapeDtypeStruct((NC, NS, ROWS_PER_SUB, D), jnp.float32),
    mesh=plsc.VectorSubcoreMesh(core_axis_name="c", subcore_axis_name="s"),
    scratch_types=[
        pltpu.VMEM((CPP, CHUNK), jnp.int32),
        pltpu.VMEM((CPP, CHUNK), jnp.int32),
        pltpu.VMEM((CPP, CHUNK), jnp.float32),
        pltpu.VMEM((CHUNK, D), jnp.float32),
        pltpu.VMEM((CHUNK, D), jnp.float32),
        pltpu.VMEM_SHARED((N, D), jnp.float32),
        pltpu.SemaphoreType.DMA,
        pltpu.SemaphoreType.DMA,
        pltpu.SemaphoreType.DMA,
        pltpu.SemaphoreType.DMA,
    ],
)
def _sc_aggregate(x_hbm, src_hbm, dst_hbm, att_hbm, out_hbm,
                  src_v, dst_v, att_v, rows0, rows1, ego_sh,
                  gsem0, gsem1, ssem0, ssem1):
    c = lax.axis_index("c")
    s = lax.axis_index("s")
    wid = s * NC + c

    zvec = jnp.zeros((16,), jnp.float32)

    def _zero_row(i, _):
        rows0[i // 8, pl.ds((i % 8) * 16, 16)] = zvec
        return 0

    lax.fori_loop(0, ZROWS * 8, _zero_row, 0)
    stripe = s * ROWS_PER_SUB
    for k in range(ROWS_PER_SUB // ZROWS):
        pltpu.sync_copy(rows0.at[pl.ds(0, ZROWS)],
                        ego_sh.at[pl.ds(stripe + k * ZROWS, ZROWS)])
    plsc.subcore_barrier()

    bufs = (rows0, rows1)
    gsems = (gsem0, gsem1)
    ssems = (ssem0, ssem1)

    def _scale(i, buf):
        def _group(g, _):
            a16 = att_v[i, pl.ds(g * 16, 16)]
            for t in range(16):
                spl = jnp.take_along_axis(a16, jnp.full((16,), t, jnp.int32),
                                          axis=0)
                e = g * 16 + t
                for j in range(D // 16):
                    buf[e, pl.ds(j * 16, 16)] = buf[e, pl.ds(j * 16, 16)] * spl
            return 0

        lax.fori_loop(0, CHUNK // 16, _group, 0)

    for p in range(PH):
        pltpu.sync_copy(src_hbm.at[wid, pl.ds(p * CPP, CPP)], src_v)
        pltpu.sync_copy(dst_hbm.at[wid, pl.ds(p * CPP, CPP)], dst_v)
        pltpu.sync_copy(att_hbm.at[wid, pl.ds(p * CPP, CPP)], att_v)

        pltpu.async_copy(x_hbm.at[src_v.at[0]], rows0, gsem0)

        def _step(i2, _):
            for b in range(2):
                i = i2 * 2 + b
                buf, obuf = bufs[b], bufs[1 - b]
                gsem, gsem_o = gsems[b], gsems[1 - b]
                ssem, ssem_o = ssems[b], ssems[1 - b]
                pltpu.make_async_copy(x_hbm.at[src_v.at[i]], buf, gsem).wait()
                @pl.when(i >= 1)
                def _():
                    pltpu.make_async_copy(
                        obuf, ego_sh.at[dst_v.at[i - 1]], ssem_o).wait()

                @pl.when(i + 1 < CPP)
                def _():
                    pltpu.async_copy(x_hbm.at[src_v.at[i + 1]], obuf, gsem_o)

                _scale(i, buf)
                pltpu.async_copy(buf, ego_sh.at[dst_v.at[i]], ssem, add=True)
            return 0

        lax.fori_loop(0, CPP // 2, _step, 0)
        pltpu.make_async_copy(rows1, ego_sh.at[dst_v.at[CPP - 1]],
                              ssem1).wait()

    plsc.subcore_barrier()

    pltpu.sync_copy(ego_sh.at[pl.ds(stripe, ROWS_PER_SUB)], out_hbm.at[c, s])


BLK = 1000


def _mlp_body(x_ref, p0_ref, p1_ref, w1_ref, b1_ref, w2_ref, b2_ref, o_ref):
    ego = p0_ref[...] + p1_ref[...]
    xv = x_ref[...]
    h1 = jnp.dot(xv + ego, w1_ref[...], preferred_element_type=jnp.float32) + b1_ref[...]
    h2 = jnp.dot(xv * ego, w2_ref[...], preferred_element_type=jnp.float32) + b2_ref[...]
    o_ref[...] = (jnp.where(h1 >= 0, h1, 0.01 * h1)
                  + jnp.where(h2 >= 0, h2, 0.01 * h2))


def _mlp(x, partials, W1, b1, W2, b2):
    grid = N // BLK
    return pl.pallas_call(
        _mlp_body,
        grid=(grid,),
        in_specs=[
            pl.BlockSpec((BLK, D), lambda i: (i, 0)),
            pl.BlockSpec((BLK, D), lambda i: (i, 0)),
            pl.BlockSpec((BLK, D), lambda i: (i + N // BLK, 0)),
            pl.BlockSpec((D, D), lambda i: (0, 0)),
            pl.BlockSpec((1, D), lambda i: (0, 0)),
            pl.BlockSpec((D, D), lambda i: (0, 0)),
            pl.BlockSpec((1, D), lambda i: (0, 0)),
        ],
        out_specs=pl.BlockSpec((BLK, D), lambda i: (i, 0)),
        out_shape=jax.ShapeDtypeStruct((N, D), jnp.float32),
    )(x, partials, partials, W1, b1, W2, b2)


def kernel(x, edge_index, attention, W1, b1, W2, b2):
    src = edge_index[0].astype(jnp.int32)
    dst = edge_index[1].astype(jnp.int32)
    pad = E_PAD - E
    src = jnp.concatenate([src, jnp.zeros((pad,), jnp.int32)])
    dst = jnp.concatenate([dst, jnp.zeros((pad,), jnp.int32)])
    att = jnp.concatenate([attention, jnp.zeros((pad,), jnp.float32)])
    src = src.reshape(NW, CPW, CHUNK)
    dst = dst.reshape(NW, CPW, CHUNK)
    att = att.reshape(NW, CPW, CHUNK)
    partials = _sc_aggregate(x, src, dst, att).reshape(NC * N, D)
    return _mlp(x, partials, W1, b1.reshape(1, D), W2, b2.reshape(1, D))
